# baseline (speedup 1.0000x reference)
import functools

import jax
import jax.numpy as jnp
from jax import lax
from jax.experimental import pallas as pl
from jax.experimental.pallas import tpu as pltpu
from jax.experimental.pallas import tpu_sc as plsc

NC = 2
NS = 16
L = 16
NW = NC * NS

B = 16384
D = 16
BPW = B // NW
NVEC = BPW // L

V = 1_000_000
LANE_TILES = (V + 127) // 128
TPW = 245
WTC = 15
WCOL = WTC * 128
NWIN = (TPW + WTC - 1) // WTC
MAXC0 = (LANE_TILES - WTC) * 128
VP = LANE_TILES * 128

_mesh = plsc.VectorSubcoreMesh(core_axis_name="c", subcore_axis_name="s")


@functools.partial(
    pl.kernel,
    out_type=[
        jax.ShapeDtypeStruct((D * VP,), jnp.float32),
        jax.ShapeDtypeStruct((D * VP,), jnp.float32),
    ],
    mesh=_mesh,
    compiler_params=pltpu.CompilerParams(
        needs_layout_passes=False, use_tc_tiling_on_sc=True
    ),
    scratch_types=[
        pltpu.VMEM((4, D, WCOL), jnp.float32),
        pltpu.SemaphoreType.DMA,
        pltpu.SemaphoreType.DMA,
    ],
)
def _relayout_kernel(uf_hbm, vf_hbm, u_lin, v_lin, buf, sem_r, sem_w):
    wid = lax.axis_index("s") * NC + lax.axis_index("c")

    def col0(w):
        c = (wid * TPW + w * WTC) * 128
        return pl.multiple_of(jnp.minimum(c, MAXC0), 128)

    def pass_for(src_hbm, dst_lin):
        def read(w, slot):
            pltpu.async_copy(
                src_hbm.at[pl.ds(0, D), pl.ds(col0(w), WCOL)],
                buf.at[slot], sem_r)

        def wait_read():
            pltpu.make_async_copy(
                src_hbm.at[pl.ds(0, D), pl.ds(0, WCOL)], buf.at[0],
                sem_r).wait()

        def write(w, slot):
            c0 = col0(w)
            for j in range(D):
                pltpu.async_copy(
                    buf.at[slot, j], dst_lin.at[pl.ds(j * VP + c0, WCOL)],
                    sem_w)

        def drain_writes():
            for j in range(D):
                pltpu.make_async_copy(
                    buf.at[0, j], dst_lin.at[pl.ds(0, WCOL)], sem_w).wait()

        read(0, 0)
        read(1, 1)

        def quad(q, carry):
            for t in range(4):
                w = 4 * q + t
                wait_read()
                if t >= 2:
                    drain_writes()
                else:
                    @pl.when(q >= 1)
                    def _():
                        drain_writes()
                read(w + 2, (t + 2) % 4)
                write(w, t)
            return carry

        lax.fori_loop(0, NWIN // 4, quad, 0)
        wait_read()
        drain_writes()
        write(NWIN - 1, 0)
        wait_read()
        drain_writes()
        drain_writes()

    pass_for(uf_hbm, u_lin)
    pass_for(vf_hbm, v_lin)


@functools.partial(
    pl.kernel,
    out_type=jax.ShapeDtypeStruct((B,), jnp.float32),
    mesh=_mesh,
    compiler_params=pltpu.CompilerParams(
        needs_layout_passes=False, use_tc_tiling_on_sc=True
    ),
    scratch_types=[
        pltpu.VMEM((BPW,), jnp.int32),
        pltpu.VMEM((BPW,), jnp.int32),
        pltpu.VMEM((D * BPW,), jnp.int32),
        pltpu.VMEM((D * BPW,), jnp.int32),
        pltpu.VMEM((D * BPW,), jnp.float32),
        pltpu.VMEM((D * BPW,), jnp.float32),
        pltpu.VMEM((BPW,), jnp.float32),
        pltpu.SemaphoreType.DMA,
        pltpu.SemaphoreType.DMA,
    ],
)
def _dot_kernel(u_idx_hbm, v_idx_hbm, u_lin, v_lin, out_hbm,
                idx_u, idx_v, offs_u, offs_v, ubuf, vbuf, outb,
                sem_u, sem_v):
    wid = lax.axis_index("s") * NC + lax.axis_index("c")
    base = wid * BPW

    pltpu.sync_copy(u_idx_hbm.at[pl.ds(base, BPW)], idx_u)
    pltpu.sync_copy(v_idx_hbm.at[pl.ds(base, BPW)], idx_v)

    def build_offsets(kv, carry):
        iu = idx_u[pl.ds(kv * L, L)]
        iv = idx_v[pl.ds(kv * L, L)]
        for j in range(D):
            offs_u[pl.ds(j * BPW + kv * L, L)] = iu + j * VP
            offs_v[pl.ds(j * BPW + kv * L, L)] = iv + j * VP
        return carry

    lax.fori_loop(0, NVEC, build_offsets, 0)

    copies = []
    for j in range(D):
        sl = pl.ds(j * BPW, BPW)
        copies.append(
            pltpu.async_copy(u_lin.at[offs_u.at[sl]], ubuf.at[sl], sem_u))
        copies.append(
            pltpu.async_copy(v_lin.at[offs_v.at[sl]], vbuf.at[sl], sem_v))
    for c in copies:
        c.wait()

    def accumulate(kv, carry):
        acc = jnp.zeros((L,), jnp.float32)
        for j in range(D):
            sl = pl.ds(j * BPW + kv * L, L)
            acc = acc + ubuf[sl] * vbuf[sl]
        outb[pl.ds(kv * L, L)] = acc
        return carry

    lax.fori_loop(0, NVEC, accumulate, 0)
    pltpu.sync_copy(outb, out_hbm.at[pl.ds(base, BPW)])


def kernel(data, user_factors, item_factors):
    u_lin, v_lin = _relayout_kernel(user_factors.T, item_factors.T)
    return _dot_kernel(data[0], data[1], u_lin, v_lin)


# speedup vs baseline: 1.0184x; 1.0184x over previous
import functools

import jax
import jax.numpy as jnp
from jax import lax
from jax.experimental import pallas as pl
from jax.experimental.pallas import tpu as pltpu
from jax.experimental.pallas import tpu_sc as plsc

NC = 2
NS = 16
L = 16
NW = NC * NS

B = 16384
D = 16
BPW = B // NW
NVEC = BPW // L

V = 1_000_000
LANE_TILES = (V + 127) // 128
TPW = 245
WTC = 15
WCOL = WTC * 128
NWIN = (TPW + WTC - 1) // WTC
MAXC0 = (LANE_TILES - WTC) * 128
VP = LANE_TILES * 128

_mesh = plsc.VectorSubcoreMesh(core_axis_name="c", subcore_axis_name="s")


@functools.partial(
    pl.kernel,
    out_type=[
        jax.ShapeDtypeStruct((D * VP,), jnp.float32),
        jax.ShapeDtypeStruct((D * VP,), jnp.float32),
    ],
    mesh=_mesh,
    compiler_params=pltpu.CompilerParams(
        needs_layout_passes=False, use_tc_tiling_on_sc=True
    ),
    scratch_types=[
        pltpu.VMEM((2, D, WCOL), jnp.float32),
        pltpu.VMEM((2, D, WCOL), jnp.float32),
        pltpu.SemaphoreType.DMA,
        pltpu.SemaphoreType.DMA,
        pltpu.SemaphoreType.DMA,
    ],
)
def _relayout_kernel(uf_hbm, vf_hbm, u_lin, v_lin, ubuf, vbuf,
                     sem_r, sem_wu, sem_wv):
    wid = lax.axis_index("s") * NC + lax.axis_index("c")

    def col0(w):
        c = (wid * TPW + w * WTC) * 128
        return pl.multiple_of(jnp.minimum(c, MAXC0), 128)

    def read(w, slot):
        c0 = col0(w)
        pltpu.async_copy(
            uf_hbm.at[pl.ds(0, D), pl.ds(c0, WCOL)], ubuf.at[slot], sem_r)
        pltpu.async_copy(
            vf_hbm.at[pl.ds(0, D), pl.ds(c0, WCOL)], vbuf.at[slot], sem_r)

    def wait_read():
        pltpu.make_async_copy(
            uf_hbm.at[pl.ds(0, D), pl.ds(0, WCOL)], ubuf.at[0], sem_r).wait()
        pltpu.make_async_copy(
            vf_hbm.at[pl.ds(0, D), pl.ds(0, WCOL)], vbuf.at[0], sem_r).wait()

    def write(w, slot):
        c0 = col0(w)
        for j in range(D):
            pltpu.async_copy(
                ubuf.at[slot, j], u_lin.at[pl.ds(j * VP + c0, WCOL)], sem_wu)
            pltpu.async_copy(
                vbuf.at[slot, j], v_lin.at[pl.ds(j * VP + c0, WCOL)], sem_wv)

    def drain_writes():
        for j in range(D):
            pltpu.make_async_copy(
                ubuf.at[0, j], u_lin.at[pl.ds(0, WCOL)], sem_wu).wait()
            pltpu.make_async_copy(
                vbuf.at[0, j], v_lin.at[pl.ds(0, WCOL)], sem_wv).wait()

    read(0, 0)

    def pair(p, carry):
        a = 2 * p
        wait_read()
        @pl.when(p >= 1)
        def _():
            drain_writes()
        read(a + 1, 1)
        write(a, 0)
        wait_read()
        drain_writes()
        read(a + 2, 0)
        write(a + 1, 1)
        return carry

    lax.fori_loop(0, NWIN // 2, pair, 0)
    wait_read()
    drain_writes()
    write(NWIN - 1, 0)
    drain_writes()


@functools.partial(
    pl.kernel,
    out_type=jax.ShapeDtypeStruct((B,), jnp.float32),
    mesh=_mesh,
    compiler_params=pltpu.CompilerParams(
        needs_layout_passes=False, use_tc_tiling_on_sc=True
    ),
    scratch_types=[
        pltpu.VMEM((BPW,), jnp.int32),
        pltpu.VMEM((BPW,), jnp.int32),
        pltpu.VMEM((D * BPW,), jnp.int32),
        pltpu.VMEM((D * BPW,), jnp.int32),
        pltpu.VMEM((D * BPW,), jnp.float32),
        pltpu.VMEM((D * BPW,), jnp.float32),
        pltpu.VMEM((BPW,), jnp.float32),
        pltpu.SemaphoreType.DMA,
        pltpu.SemaphoreType.DMA,
    ],
)
def _dot_kernel(u_idx_hbm, v_idx_hbm, u_lin, v_lin, out_hbm,
                idx_u, idx_v, offs_u, offs_v, ubuf, vbuf, outb,
                sem_u, sem_v):
    wid = lax.axis_index("s") * NC + lax.axis_index("c")
    base = wid * BPW

    pltpu.sync_copy(u_idx_hbm.at[pl.ds(base, BPW)], idx_u)
    pltpu.sync_copy(v_idx_hbm.at[pl.ds(base, BPW)], idx_v)

    def build_offsets(kv, carry):
        iu = idx_u[pl.ds(kv * L, L)]
        iv = idx_v[pl.ds(kv * L, L)]
        for j in range(D):
            offs_u[pl.ds(j * BPW + kv * L, L)] = iu + j * VP
            offs_v[pl.ds(j * BPW + kv * L, L)] = iv + j * VP
        return carry

    lax.fori_loop(0, NVEC, build_offsets, 0)

    copies = []
    for j in range(D):
        sl = pl.ds(j * BPW, BPW)
        copies.append(
            pltpu.async_copy(u_lin.at[offs_u.at[sl]], ubuf.at[sl], sem_u))
        copies.append(
            pltpu.async_copy(v_lin.at[offs_v.at[sl]], vbuf.at[sl], sem_v))
    for c in copies:
        c.wait()

    def accumulate(kv, carry):
        acc = jnp.zeros((L,), jnp.float32)
        for j in range(D):
            sl = pl.ds(j * BPW + kv * L, L)
            acc = acc + ubuf[sl] * vbuf[sl]
        outb[pl.ds(kv * L, L)] = acc
        return carry

    lax.fori_loop(0, NVEC, accumulate, 0)
    pltpu.sync_copy(outb, out_hbm.at[pl.ds(base, BPW)])


def kernel(data, user_factors, item_factors):
    u_lin, v_lin = _relayout_kernel(user_factors.T, item_factors.T)
    return _dot_kernel(data[0], data[1], u_lin, v_lin)


# speedup vs baseline: 1.0324x; 1.0138x over previous
import functools

import jax
import jax.numpy as jnp
from jax import lax
from jax.experimental import pallas as pl
from jax.experimental.pallas import tpu as pltpu
from jax.experimental.pallas import tpu_sc as plsc

NC = 2
NS = 16
L = 16
NW = NC * NS

B = 16384
D = 16
BPW = B // NW
NVEC = BPW // L

V = 1_000_000
LANE_TILES = (V + 127) // 128
TPW = 245
WTC = 7
WCOL = WTC * 128
NWIN = (TPW + WTC - 1) // WTC
MAXC0 = (LANE_TILES - WTC) * 128
VP = LANE_TILES * 128

_mesh = plsc.VectorSubcoreMesh(core_axis_name="c", subcore_axis_name="s")


@functools.partial(
    pl.kernel,
    out_type=[
        jax.ShapeDtypeStruct((D * VP,), jnp.float32),
        jax.ShapeDtypeStruct((D * VP,), jnp.float32),
    ],
    mesh=_mesh,
    compiler_params=pltpu.CompilerParams(
        needs_layout_passes=False, use_tc_tiling_on_sc=True
    ),
    scratch_types=[
        pltpu.VMEM((4, D, WCOL), jnp.float32),
        pltpu.VMEM((4, D, WCOL), jnp.float32),
        pltpu.SemaphoreType.DMA,
        pltpu.SemaphoreType.DMA,
        pltpu.SemaphoreType.DMA,
    ],
)
def _relayout_kernel(uf_hbm, vf_hbm, u_lin, v_lin, ubuf, vbuf,
                     sem_r, sem_wu, sem_wv):
    wid = lax.axis_index("s") * NC + lax.axis_index("c")

    def col0(w):
        c = (wid * TPW + w * WTC) * 128
        return pl.multiple_of(jnp.minimum(c, MAXC0), 128)

    def read(w, slot):
        c0 = col0(w)
        pltpu.async_copy(
            uf_hbm.at[pl.ds(0, D), pl.ds(c0, WCOL)], ubuf.at[slot], sem_r)
        pltpu.async_copy(
            vf_hbm.at[pl.ds(0, D), pl.ds(c0, WCOL)], vbuf.at[slot], sem_r)

    def wait_read():
        pltpu.make_async_copy(
            uf_hbm.at[pl.ds(0, D), pl.ds(0, WCOL)], ubuf.at[0], sem_r).wait()
        pltpu.make_async_copy(
            vf_hbm.at[pl.ds(0, D), pl.ds(0, WCOL)], vbuf.at[0], sem_r).wait()

    def write(w, slot):
        c0 = col0(w)
        for j in range(D):
            pltpu.async_copy(
                ubuf.at[slot, j], u_lin.at[pl.ds(j * VP + c0, WCOL)], sem_wu)
            pltpu.async_copy(
                vbuf.at[slot, j], v_lin.at[pl.ds(j * VP + c0, WCOL)], sem_wv)

    def drain_writes():
        for j in range(D):
            pltpu.make_async_copy(
                ubuf.at[0, j], u_lin.at[pl.ds(0, WCOL)], sem_wu).wait()
            pltpu.make_async_copy(
                vbuf.at[0, j], v_lin.at[pl.ds(0, WCOL)], sem_wv).wait()

    read(0, 0)
    read(1, 1)

    def quad(q, carry):
        for t in range(4):
            w = 4 * q + t
            wait_read()
            if t >= 2:
                drain_writes()
            else:
                @pl.when(q >= 1)
                def _():
                    drain_writes()
            read(w + 2, (t + 2) % 4)
            write(w, t)
        return carry

    lax.fori_loop(0, NWIN // 4, quad, 0)
    for w in (NWIN - 3, NWIN - 2, NWIN - 1):
        wait_read()
        drain_writes()
        read(w + 2, (w + 2) % 4)
        write(w, w % 4)
    wait_read()
    wait_read()
    drain_writes()
    drain_writes()


@functools.partial(
    pl.kernel,
    out_type=jax.ShapeDtypeStruct((B,), jnp.float32),
    mesh=_mesh,
    compiler_params=pltpu.CompilerParams(
        needs_layout_passes=False, use_tc_tiling_on_sc=True
    ),
    scratch_types=[
        pltpu.VMEM((BPW,), jnp.int32),
        pltpu.VMEM((BPW,), jnp.int32),
        pltpu.VMEM((D * BPW,), jnp.int32),
        pltpu.VMEM((D * BPW,), jnp.int32),
        pltpu.VMEM((D * BPW,), jnp.float32),
        pltpu.VMEM((D * BPW,), jnp.float32),
        pltpu.VMEM((BPW,), jnp.float32),
        pltpu.SemaphoreType.DMA,
        pltpu.SemaphoreType.DMA,
    ],
)
def _dot_kernel(u_idx_hbm, v_idx_hbm, u_lin, v_lin, out_hbm,
                idx_u, idx_v, offs_u, offs_v, ubuf, vbuf, outb,
                sem_u, sem_v):
    wid = lax.axis_index("s") * NC + lax.axis_index("c")
    base = wid * BPW

    pltpu.sync_copy(u_idx_hbm.at[pl.ds(base, BPW)], idx_u)
    pltpu.sync_copy(v_idx_hbm.at[pl.ds(base, BPW)], idx_v)

    def build_offsets(kv, carry):
        iu = idx_u[pl.ds(kv * L, L)]
        iv = idx_v[pl.ds(kv * L, L)]
        for j in range(D):
            offs_u[pl.ds(j * BPW + kv * L, L)] = iu + j * VP
            offs_v[pl.ds(j * BPW + kv * L, L)] = iv + j * VP
        return carry

    lax.fori_loop(0, NVEC, build_offsets, 0)

    copies = []
    for j in range(D):
        sl = pl.ds(j * BPW, BPW)
        copies.append(
            pltpu.async_copy(u_lin.at[offs_u.at[sl]], ubuf.at[sl], sem_u))
        copies.append(
            pltpu.async_copy(v_lin.at[offs_v.at[sl]], vbuf.at[sl], sem_v))
    for c in copies:
        c.wait()

    def accumulate(kv, carry):
        acc = jnp.zeros((L,), jnp.float32)
        for j in range(D):
            sl = pl.ds(j * BPW + kv * L, L)
            acc = acc + ubuf[sl] * vbuf[sl]
        outb[pl.ds(kv * L, L)] = acc
        return carry

    lax.fori_loop(0, NVEC, accumulate, 0)
    pltpu.sync_copy(outb, out_hbm.at[pl.ds(base, BPW)])


def kernel(data, user_factors, item_factors):
    u_lin, v_lin = _relayout_kernel(user_factors.T, item_factors.T)
    return _dot_kernel(data[0], data[1], u_lin, v_lin)
